# baseline (speedup 1.0000x reference)
import jax
import jax.numpy as jnp
from jax.experimental import pallas as pl
from jax.experimental.pallas import tpu as pltpu


def _copy_kernel(shift_ref, x_ref, o_ref):
    o_ref[...] = x_ref[...]


def kernel(x, shift):
    b, f, t = x.shape
    s = jnp.reshape(shift.astype(jnp.int32) % f, (1,))
    x2 = x.reshape(b, f * t)
    grid = (f,)
    out2 = pl.pallas_call(
        _copy_kernel,
        out_shape=jax.ShapeDtypeStruct((b, f * t), x.dtype),
        grid_spec=pltpu.PrefetchScalarGridSpec(
            num_scalar_prefetch=1,
            grid=grid,
            in_specs=[pl.BlockSpec((b, t), lambda j, s: (0, (j - s[0]) % f))],
            out_specs=pl.BlockSpec((b, t), lambda j, s: (0, j)),
        ),
        compiler_params=pltpu.CompilerParams(
            dimension_semantics=("arbitrary",),
            vmem_limit_bytes=60 * 1024 * 1024,
        ),
    )(s, x2)
    return out2.reshape(b, f, t)


# speedup vs baseline: 4.7617x; 4.7617x over previous
import jax
import jax.numpy as jnp
from jax import lax
from jax.experimental import pallas as pl
from jax.experimental.pallas import tpu as pltpu

_MAX_STATIC = 5


def _roll_kernel(shift_ref, x_ref, o_ref):
    f = x_ref.shape[1]
    raw = shift_ref[0]
    idx = jnp.where(jnp.abs(raw) <= _MAX_STATIC, raw + _MAX_STATIC,
                    2 * _MAX_STATIC + 1)

    def static_case(k):
        def body():
            o_ref[...] = pltpu.roll(x_ref[...], k % f, axis=1)
        return body

    def dynamic_case():
        o_ref[...] = pltpu.roll(x_ref[...], shift_ref[1], axis=1)

    branches = [static_case(k) for k in range(-_MAX_STATIC, _MAX_STATIC + 1)]
    branches.append(dynamic_case)
    lax.switch(idx, branches)


def kernel(x, shift):
    b, f, t = x.shape
    raw = shift.astype(jnp.int32)
    s = jnp.stack([raw, raw % f])
    bb = 16
    grid = (b // bb,)
    return pl.pallas_call(
        _roll_kernel,
        out_shape=jax.ShapeDtypeStruct((b, f, t), x.dtype),
        grid_spec=pltpu.PrefetchScalarGridSpec(
            num_scalar_prefetch=1,
            grid=grid,
            in_specs=[pl.BlockSpec((bb, f, t), lambda i, s: (i, 0, 0))],
            out_specs=pl.BlockSpec((bb, f, t), lambda i, s: (i, 0, 0)),
        ),
        compiler_params=pltpu.CompilerParams(
            dimension_semantics=("parallel",),
            vmem_limit_bytes=60 * 1024 * 1024,
        ),
    )(s, x)
